# baseline (speedup 1.0000x reference)
import functools

import jax
import jax.numpy as jnp
import numpy as np
from jax import lax
from jax.experimental import pallas as pl
from jax.experimental.pallas import tpu as pltpu
from jax.experimental.pallas import tpu_sc as plsc

N = 10000
E = 320000
D = 128
DELTA = float(np.log(2.0))

NC = 2
NS = 16
NW = NC * NS
NPT = 320
NPAD = NW * NPT
NCORE = NS * NPT
EBLK = 4000
NBLK = E // EBLK
CH = 128
DUMP = NCORE
BIG = 3.0e38

_mesh = plsc.VectorSubcoreMesh(core_axis_name="c", subcore_axis_name="s")


@functools.partial(
    pl.kernel,
    mesh=_mesh,
    compiler_params=pltpu.CompilerParams(needs_layout_passes=False),
    out_type=[
        jax.ShapeDtypeStruct((NW, NBLK, EBLK), jnp.int32),
        jax.ShapeDtypeStruct((NW, NBLK, EBLK), jnp.int32),
        jax.ShapeDtypeStruct((NW, 128), jnp.int32),
        jax.ShapeDtypeStruct((NW, 320), jnp.float32),
        jax.ShapeDtypeStruct((NW, 320), jnp.float32),
    ],
    scratch_types=[
        pltpu.VMEM((EBLK,), jnp.int32),
        pltpu.VMEM((EBLK,), jnp.int32),
        pltpu.VMEM((EBLK + 160,), jnp.int32),
        pltpu.VMEM((EBLK + 160,), jnp.int32),
        pltpu.VMEM((128,), jnp.int32),
        pltpu.VMEM((320,), jnp.float32),
        pltpu.VMEM((320,), jnp.float32),
    ],
)
def _setup(esrc_hbm, edst_hbm, srcl_hbm, ldl_hbm, cnts_hbm, cnt_hbm, deg_hbm,
           src_b, dst_b, cs_b, cl_b, cnts_b, cnt_a, deg_a):
    c = lax.axis_index("c")
    s = lax.axis_index("s")
    w = c * NS + s
    base = w * NPT
    zf = jnp.zeros((16,), jnp.float32)
    zi = jnp.zeros((16,), jnp.int32)
    iota16 = lax.iota(jnp.int32, 16)

    def zrow(i, _):
        cnt_a[pl.ds(i * 16, 16)] = zf
        deg_a[pl.ds(i * 16, 16)] = zf
        return 0
    lax.fori_loop(0, 20, zrow, 0)

    def zcnts(i, _):
        cnts_b[pl.ds(i * 16, 16)] = zi
        return 0
    lax.fori_loop(0, 8, zcnts, 0)

    def block(b, _):
        pltpu.sync_copy(esrc_hbm.at[pl.ds(b * EBLK, EBLK)], src_b)
        pltpu.sync_copy(edst_hbm.at[pl.ds(b * EBLK, EBLK)], dst_b)

        def vreg(k, cnt):
            vs = src_b[pl.ds(k * 16, 16)]
            vd = dst_b[pl.ds(k * 16, 16)]
            own = (vd >= base) & (vd < base + NPT)
            ld = jnp.clip(vd - base, 0, NPT - 1)
            key = iota16 + jnp.where(own, 0, 16)
            _, vs_s = plsc.sort_key_val(key, vs)
            _, ld_s = plsc.sort_key_val(key, ld)
            cs_b[pl.ds(cnt, 16)] = vs_s
            cl_b[pl.ds(cnt, 16)] = ld_s
            plsc.addupdate_scatter(cnt_a, [ld], jnp.where(own, 1.0, 0.0))
            owns = (vs >= base) & (vs < base + NPT)
            ls = jnp.clip(vs - base, 0, NPT - 1)
            plsc.addupdate_scatter(deg_a, [ls], jnp.where(owns, 1.0, 0.0))
            pc = plsc.all_reduce_population_count(own)[0]
            return cnt + pc
        cnt = lax.fori_loop(0, EBLK // 16, vreg, 0)

        def zpad(i, _):
            cs_b[pl.ds(cnt + i * 16, 16)] = zi
            cl_b[pl.ds(cnt + i * 16, 16)] = zi
            return 0
        lax.fori_loop(0, 9, zpad, 0)
        plsc.store_scatter(cnts_b,
                           [jnp.where(iota16 == 0, b, 127)],
                           jnp.full((16,), cnt, jnp.int32))

        nch = (cnt + CH - 1) // CH

        def drain(i, _):
            pltpu.sync_copy(cs_b.at[pl.ds(i * CH, CH)],
                            srcl_hbm.at[w, b, pl.ds(i * CH, CH)])
            pltpu.sync_copy(cl_b.at[pl.ds(i * CH, CH)],
                            ldl_hbm.at[w, b, pl.ds(i * CH, CH)])
            return 0
        lax.fori_loop(0, nch, drain, 0)
        return 0
    lax.fori_loop(0, NBLK, block, 0)

    pltpu.sync_copy(cnts_b, cnts_hbm.at[w])
    pltpu.sync_copy(cnt_a, cnt_hbm.at[w])
    pltpu.sync_copy(deg_a, deg_hbm.at[w])


@functools.partial(
    pl.kernel,
    mesh=_mesh,
    compiler_params=pltpu.CompilerParams(needs_layout_passes=False),
    out_type=[
        jax.ShapeDtypeStruct((NPAD, D), jnp.float32),
        jax.ShapeDtypeStruct((NPAD, D), jnp.float32),
    ],
    scratch_types=[
        pltpu.VMEM((CH,), jnp.int32),
        pltpu.VMEM((CH + 16,), jnp.int32),
        pltpu.VMEM((CH,), jnp.int32),
        pltpu.VMEM((CH, D), jnp.float32),
        pltpu.VMEM((CH, D), jnp.float32),
        pltpu.VMEM((144,), jnp.int32),
        pltpu.VMEM_SHARED((NCORE + 16, D), jnp.float32),
        pltpu.VMEM_SHARED((NCORE + 16, D), jnp.float32),
        pltpu.SemaphoreType.DMA,
    ],
)
def _scatter_sums(h_hbm, srcl_hbm, ldl_hbm, cnts_hbm, zeros_hbm,
                  s_out, s2_out,
                  sidx, lidx, gidx, msg, sq, cnts_b, ssum, ssq, sem):
    c = lax.axis_index("c")
    s = lax.axis_index("s")
    w = c * NS + s
    iota16 = lax.iota(jnp.int32, 16)

    @pl.when(s == 0)
    def _():
        pltpu.sync_copy(zeros_hbm, ssum.at[pl.ds(0, NCORE)])
        pltpu.sync_copy(zeros_hbm, ssq.at[pl.ds(0, NCORE)])
    plsc.subcore_barrier()

    pltpu.sync_copy(cnts_hbm.at[w], cnts_b.at[pl.ds(0, 128)])

    def block(b, _):
        cnt = cnts_b[pl.ds(b, 16)][0]
        nch = (cnt + CH - 1) // CH

        def chunk(i, _):
            pltpu.sync_copy(srcl_hbm.at[w, b, pl.ds(i * CH, CH)], sidx)
            pltpu.sync_copy(ldl_hbm.at[w, b, pl.ds(i * CH, CH)], lidx.at[pl.ds(0, CH)])
            pltpu.async_copy(h_hbm.at[sidx], msg, sem).wait()
            nb = jnp.minimum(cnt - i * CH, CH)
            for v in range(8):
                lv = lidx[pl.ds(v * 16, 16)]
                lane = iota16 + (v * 16)
                gidx[pl.ds(v * 16, 16)] = jnp.where(lane < nb, lv + s * NPT, DUMP)

            def row(e, _):
                for j in range(8):
                    m = msg[e, pl.ds(j * 16, 16)]
                    sq[e, pl.ds(j * 16, 16)] = m * m
                return 0
            lax.fori_loop(0, nb, row, 0)

            pltpu.sync_copy(msg, ssum.at[gidx], add=True)
            pltpu.sync_copy(sq, ssq.at[gidx], add=True)
            return 0
        lax.fori_loop(0, nch, chunk, 0)
        return 0
    lax.fori_loop(0, NBLK, block, 0)

    pltpu.sync_copy(ssum.at[pl.ds(s * NPT, NPT)], s_out.at[pl.ds(w * NPT, NPT)])
    pltpu.sync_copy(ssq.at[pl.ds(s * NPT, NPT)], s2_out.at[pl.ds(w * NPT, NPT)])


@functools.partial(
    pl.kernel,
    mesh=_mesh,
    compiler_params=pltpu.CompilerParams(needs_layout_passes=False),
    out_type=[
        jax.ShapeDtypeStruct((NPAD, D), jnp.float32),
        jax.ShapeDtypeStruct((NPAD, D), jnp.float32),
    ],
    scratch_types=[
        pltpu.VMEM((CH,), jnp.int32),
        pltpu.VMEM((CH + 16,), jnp.int32),
        pltpu.VMEM((CH, D), jnp.float32),
        pltpu.VMEM((NPT, D), jnp.float32),
        pltpu.VMEM((NPT, D), jnp.float32),
        pltpu.VMEM((144,), jnp.int32),
        pltpu.SemaphoreType.DMA,
    ],
)
def _scatter_minmax(h_hbm, srcl_hbm, ldl_hbm, cnts_hbm,
                    mx_out, mn_out,
                    sidx, lidx, msg, mxa, mna, cnts_b, sem):
    c = lax.axis_index("c")
    s = lax.axis_index("s")
    w = c * NS + s
    vbig = jnp.full((16,), BIG, jnp.float32)

    def initrow(i, _):
        for j in range(8):
            mxa[i, pl.ds(j * 16, 16)] = -vbig
            mna[i, pl.ds(j * 16, 16)] = vbig
        return 0
    lax.fori_loop(0, NPT, initrow, 0)

    pltpu.sync_copy(cnts_hbm.at[w], cnts_b.at[pl.ds(0, 128)])

    def block(b, _):
        cnt = cnts_b[pl.ds(b, 16)][0]
        nch = (cnt + CH - 1) // CH

        def chunk(i, _):
            pltpu.sync_copy(srcl_hbm.at[w, b, pl.ds(i * CH, CH)], sidx)
            pltpu.sync_copy(ldl_hbm.at[w, b, pl.ds(i * CH, CH)], lidx.at[pl.ds(0, CH)])
            pltpu.async_copy(h_hbm.at[sidx], msg, sem).wait()
            nb = jnp.minimum(cnt - i * CH, CH)

            def edge(e, _):
                ld = lidx[pl.ds(e, 16)][0]
                for j in range(8):
                    m = msg[e, pl.ds(j * 16, 16)]
                    a = mxa[ld, pl.ds(j * 16, 16)]
                    mxa[ld, pl.ds(j * 16, 16)] = jnp.maximum(a, m)
                    g = mna[ld, pl.ds(j * 16, 16)]
                    mna[ld, pl.ds(j * 16, 16)] = jnp.minimum(g, m)
                return 0
            lax.fori_loop(0, nb, edge, 0)
            return 0
        lax.fori_loop(0, nch, chunk, 0)
        return 0
    lax.fori_loop(0, NBLK, block, 0)

    pltpu.sync_copy(mxa, mx_out.at[pl.ds(w * NPT, NPT)])
    pltpu.sync_copy(mna, mn_out.at[pl.ds(w * NPT, NPT)])


BN = 2560


def _finalize_body(s_ref, s2_ref, mx_ref, mn_ref, cnt_ref, deg_ref,
                   hprev_ref, wp_ref, b_ref, out_ref):
    sm = s_ref[...]
    s2 = s2_ref[...]
    mx = mx_ref[...]
    mn = mn_ref[...]
    cnt = cnt_ref[:, 0:1]
    has = cnt > 0.0
    cnt_c = jnp.maximum(cnt, 1.0)
    mean = sm / cnt_c
    var = jnp.maximum(s2 / cnt_c - mean * mean, 0.0)
    std = jnp.sqrt(var + 1e-5)
    zero = jnp.zeros_like(mean)
    mean = jnp.where(has, mean, zero)
    std = jnp.where(has, std, jnp.full_like(std, float(np.sqrt(np.float32(1e-5)))))
    mx = jnp.where(has, mx, zero)
    mn = jnp.where(has, mn, zero)
    agg = jnp.concatenate([mean, mx, mn, std], axis=1)
    y = jnp.dot(agg, wp_ref[...], preferred_element_type=jnp.float32)
    deg = deg_ref[:, 0:1]
    ldeg = jnp.log(jnp.maximum(deg, 1.0) + 1.0)
    amp = ldeg * (1.0 / DELTA)
    att = DELTA / ldeg
    out = y[:, 0:128] + amp * y[:, 128:256] + att * y[:, 256:384]
    out_ref[...] = out + b_ref[...] + hprev_ref[...]


_finalize = pl.pallas_call(
    _finalize_body,
    grid=(NPAD // BN,),
    in_specs=[
        pl.BlockSpec((BN, 128), lambda i: (i, 0)),
        pl.BlockSpec((BN, 128), lambda i: (i, 0)),
        pl.BlockSpec((BN, 128), lambda i: (i, 0)),
        pl.BlockSpec((BN, 128), lambda i: (i, 0)),
        pl.BlockSpec((BN, 16), lambda i: (i, 0)),
        pl.BlockSpec((BN, 16), lambda i: (i, 0)),
        pl.BlockSpec((BN, 128), lambda i: (i, 0)),
        pl.BlockSpec((512, 384), lambda i: (0, 0)),
        pl.BlockSpec((1, 128), lambda i: (0, 0)),
    ],
    out_specs=pl.BlockSpec((BN, 128), lambda i: (i, 0)),
    out_shape=jax.ShapeDtypeStruct((NPAD, D), jnp.float32),
)


def kernel(x, edge_index, W0, b0, W1, b1, W2, b2):
    srcl, ldl, cnts, cnt_t, deg_t = _setup(edge_index[0], edge_index[1])
    cnt = cnt_t.reshape(-1)
    deg = deg_t.reshape(-1)
    cnt16 = jnp.broadcast_to(cnt[:, None], (NPAD, 16))
    deg16 = jnp.broadcast_to(deg[:, None], (NPAD, 16))
    zeros = jnp.zeros((NCORE, D), jnp.float32)
    h = jnp.concatenate([x, jnp.zeros((NPAD - N, D), x.dtype)], axis=0)
    for W, b in ((W0, b0), (W1, b1), (W2, b2)):
        sm, s2 = _scatter_sums(h, srcl, ldl, cnts, zeros)
        mx, mn = _scatter_minmax(h, srcl, ldl, cnts)
        wp = jnp.concatenate([W[0:512], W[512:1024], W[1024:1536]], axis=1)
        h = _finalize(sm, s2, mx, mn, cnt16, deg16, h, wp, b.reshape(1, 128))
    return h[:N]


# speedup vs baseline: 1.0060x; 1.0060x over previous
import functools

import jax
import jax.numpy as jnp
import numpy as np
from jax import lax
from jax.experimental import pallas as pl
from jax.experimental.pallas import tpu as pltpu
from jax.experimental.pallas import tpu_sc as plsc

N = 10000
E = 320000
D = 128
DELTA = float(np.log(2.0))

NC = 2
NS = 16
NW = NC * NS
NPT = 320
NPAD = NW * NPT
NCORE = NS * NPT
EBLK = 4000
NBLK = E // EBLK
CH = 128
DUMP = NCORE
BIG = 3.0e38

_mesh = plsc.VectorSubcoreMesh(core_axis_name="c", subcore_axis_name="s")


@functools.partial(
    pl.kernel,
    mesh=_mesh,
    compiler_params=pltpu.CompilerParams(needs_layout_passes=False),
    out_type=[
        jax.ShapeDtypeStruct((NW, NBLK, EBLK), jnp.int32),
        jax.ShapeDtypeStruct((NW, NBLK, EBLK), jnp.int32),
        jax.ShapeDtypeStruct((NW, 128), jnp.int32),
        jax.ShapeDtypeStruct((NW, 320), jnp.float32),
        jax.ShapeDtypeStruct((NW, 320), jnp.float32),
    ],
    scratch_types=[
        pltpu.VMEM((EBLK,), jnp.int32),
        pltpu.VMEM((EBLK,), jnp.int32),
        pltpu.VMEM((EBLK + 160,), jnp.int32),
        pltpu.VMEM((EBLK + 160,), jnp.int32),
        pltpu.VMEM((128,), jnp.int32),
        pltpu.VMEM((320,), jnp.float32),
        pltpu.VMEM((320,), jnp.float32),
    ],
)
def _setup(esrc_hbm, edst_hbm, srcl_hbm, ldl_hbm, cnts_hbm, cnt_hbm, deg_hbm,
           src_b, dst_b, cs_b, cl_b, cnts_b, cnt_a, deg_a):
    c = lax.axis_index("c")
    s = lax.axis_index("s")
    w = c * NS + s
    base = w * NPT
    zf = jnp.zeros((16,), jnp.float32)
    zi = jnp.zeros((16,), jnp.int32)
    iota16 = lax.iota(jnp.int32, 16)

    def zrow(i, _):
        cnt_a[pl.ds(i * 16, 16)] = zf
        deg_a[pl.ds(i * 16, 16)] = zf
        return 0
    lax.fori_loop(0, 20, zrow, 0)

    def zcnts(i, _):
        cnts_b[pl.ds(i * 16, 16)] = zi
        return 0
    lax.fori_loop(0, 8, zcnts, 0)

    def block(b, _):
        pltpu.sync_copy(esrc_hbm.at[pl.ds(b * EBLK, EBLK)], src_b)
        pltpu.sync_copy(edst_hbm.at[pl.ds(b * EBLK, EBLK)], dst_b)

        def vreg(k, cnt):
            vs = src_b[pl.ds(k * 16, 16)]
            vd = dst_b[pl.ds(k * 16, 16)]
            own = (vd >= base) & (vd < base + NPT)
            ld = jnp.clip(vd - base, 0, NPT - 1)
            key = iota16 + jnp.where(own, 0, 16)
            _, vs_s = plsc.sort_key_val(key, vs)
            _, ld_s = plsc.sort_key_val(key, ld)
            cs_b[pl.ds(cnt, 16)] = vs_s
            cl_b[pl.ds(cnt, 16)] = ld_s
            plsc.addupdate_scatter(cnt_a, [ld], jnp.where(own, 1.0, 0.0))
            owns = (vs >= base) & (vs < base + NPT)
            ls = jnp.clip(vs - base, 0, NPT - 1)
            plsc.addupdate_scatter(deg_a, [ls], jnp.where(owns, 1.0, 0.0))
            pc = plsc.all_reduce_population_count(own)[0]
            return cnt + pc
        cnt = lax.fori_loop(0, EBLK // 16, vreg, 0)

        def zpad(i, _):
            cs_b[pl.ds(cnt + i * 16, 16)] = zi
            cl_b[pl.ds(cnt + i * 16, 16)] = zi
            return 0
        lax.fori_loop(0, 9, zpad, 0)
        plsc.store_scatter(cnts_b,
                           [jnp.where(iota16 == 0, b, 127)],
                           jnp.full((16,), cnt, jnp.int32))

        nch = (cnt + CH - 1) // CH

        def drain(i, _):
            pltpu.sync_copy(cs_b.at[pl.ds(i * CH, CH)],
                            srcl_hbm.at[w, b, pl.ds(i * CH, CH)])
            pltpu.sync_copy(cl_b.at[pl.ds(i * CH, CH)],
                            ldl_hbm.at[w, b, pl.ds(i * CH, CH)])
            return 0
        lax.fori_loop(0, nch, drain, 0)
        return 0
    lax.fori_loop(0, NBLK, block, 0)

    pltpu.sync_copy(cnts_b, cnts_hbm.at[w])
    pltpu.sync_copy(cnt_a, cnt_hbm.at[w])
    pltpu.sync_copy(deg_a, deg_hbm.at[w])


@functools.partial(
    pl.kernel,
    mesh=_mesh,
    compiler_params=pltpu.CompilerParams(needs_layout_passes=False),
    out_type=[
        jax.ShapeDtypeStruct((NPAD, D), jnp.float32),
        jax.ShapeDtypeStruct((NPAD, D), jnp.float32),
    ],
    scratch_types=[
        pltpu.VMEM((CH,), jnp.int32),
        pltpu.VMEM((CH + 16,), jnp.int32),
        pltpu.VMEM((CH,), jnp.int32),
        pltpu.VMEM((CH, D), jnp.float32),
        pltpu.VMEM((CH, D), jnp.float32),
        pltpu.VMEM((144,), jnp.int32),
        pltpu.VMEM_SHARED((NCORE + 16, D), jnp.float32),
        pltpu.VMEM_SHARED((NCORE + 16, D), jnp.float32),
        pltpu.SemaphoreType.DMA,
        pltpu.SemaphoreType.DMA,
    ],
)
def _scatter_sums(h_hbm, h2_hbm, srcl_hbm, ldl_hbm, cnts_hbm, zeros_hbm,
                  s_out, s2_out,
                  sidx, lidx, gidx, msg, sq, cnts_b, ssum, ssq, sem, sem2):
    c = lax.axis_index("c")
    s = lax.axis_index("s")
    w = c * NS + s
    iota16 = lax.iota(jnp.int32, 16)

    @pl.when(s == 0)
    def _():
        pltpu.sync_copy(zeros_hbm, ssum.at[pl.ds(0, NCORE)])
        pltpu.sync_copy(zeros_hbm, ssq.at[pl.ds(0, NCORE)])
    plsc.subcore_barrier()

    pltpu.sync_copy(cnts_hbm.at[w], cnts_b.at[pl.ds(0, 128)])

    def block(b, _):
        cnt = cnts_b[pl.ds(b, 16)][0]
        nch = (cnt + CH - 1) // CH

        def chunk(i, _):
            pltpu.sync_copy(srcl_hbm.at[w, b, pl.ds(i * CH, CH)], sidx)
            pltpu.sync_copy(ldl_hbm.at[w, b, pl.ds(i * CH, CH)], lidx.at[pl.ds(0, CH)])
            cp1 = pltpu.async_copy(h_hbm.at[sidx], msg, sem)
            cp2 = pltpu.async_copy(h2_hbm.at[sidx], sq, sem2)
            nb = jnp.minimum(cnt - i * CH, CH)
            for v in range(8):
                lv = lidx[pl.ds(v * 16, 16)]
                lane = iota16 + (v * 16)
                gidx[pl.ds(v * 16, 16)] = jnp.where(lane < nb, lv + s * NPT, DUMP)
            cp1.wait()
            cp2.wait()

            pltpu.sync_copy(msg, ssum.at[gidx], add=True)
            pltpu.sync_copy(sq, ssq.at[gidx], add=True)
            return 0
        lax.fori_loop(0, nch, chunk, 0)
        return 0
    lax.fori_loop(0, NBLK, block, 0)

    pltpu.sync_copy(ssum.at[pl.ds(s * NPT, NPT)], s_out.at[pl.ds(w * NPT, NPT)])
    pltpu.sync_copy(ssq.at[pl.ds(s * NPT, NPT)], s2_out.at[pl.ds(w * NPT, NPT)])


@functools.partial(
    pl.kernel,
    mesh=_mesh,
    compiler_params=pltpu.CompilerParams(needs_layout_passes=False),
    out_type=[
        jax.ShapeDtypeStruct((NPAD, D), jnp.float32),
        jax.ShapeDtypeStruct((NPAD, D), jnp.float32),
    ],
    scratch_types=[
        pltpu.VMEM((CH,), jnp.int32),
        pltpu.VMEM((CH + 16,), jnp.int32),
        pltpu.VMEM((CH, D), jnp.float32),
        pltpu.VMEM((NPT + 8, D), jnp.float32),
        pltpu.VMEM((NPT + 8, D), jnp.float32),
        pltpu.VMEM((144,), jnp.int32),
        pltpu.SemaphoreType.DMA,
    ],
)
def _scatter_minmax(h_hbm, srcl_hbm, ldl_hbm, cnts_hbm,
                    mx_out, mn_out,
                    sidx, lidx, msg, mxa, mna, cnts_b, sem):
    c = lax.axis_index("c")
    s = lax.axis_index("s")
    w = c * NS + s
    vbig = jnp.full((16,), BIG, jnp.float32)
    iota16 = lax.iota(jnp.int32, 16)

    def initrow(i, _):
        for j in range(8):
            mxa[i, pl.ds(j * 16, 16)] = -vbig
            mna[i, pl.ds(j * 16, 16)] = vbig
        return 0
    lax.fori_loop(0, NPT + 1, initrow, 0)

    pltpu.sync_copy(cnts_hbm.at[w], cnts_b.at[pl.ds(0, 128)])

    def block(b, _):
        cnt = cnts_b[pl.ds(b, 16)][0]
        nch = (cnt + CH - 1) // CH

        def chunk(i, _):
            pltpu.sync_copy(srcl_hbm.at[w, b, pl.ds(i * CH, CH)], sidx)
            pltpu.sync_copy(ldl_hbm.at[w, b, pl.ds(i * CH, CH)], lidx.at[pl.ds(0, CH)])
            pltpu.async_copy(h_hbm.at[sidx], msg, sem).wait()
            nb = jnp.minimum(cnt - i * CH, CH)

            def group(g, _):
                lv = lidx[pl.ds(g * 16, 16)]
                lane = iota16 + g * 16
                lvp = jnp.where(lane < nb, lv, NPT)
                for t in range(16):
                    ld = lvp[t]
                    e = g * 16 + t
                    ms = [msg[e, pl.ds(j * 16, 16)] for j in range(8)]
                    ax = [mxa[ld, pl.ds(j * 16, 16)] for j in range(8)]
                    an = [mna[ld, pl.ds(j * 16, 16)] for j in range(8)]
                    for j in range(8):
                        mxa[ld, pl.ds(j * 16, 16)] = jnp.maximum(ax[j], ms[j])
                    for j in range(8):
                        mna[ld, pl.ds(j * 16, 16)] = jnp.minimum(an[j], ms[j])
                return 0
            lax.fori_loop(0, CH // 16, group, 0)
            return 0
        lax.fori_loop(0, nch, chunk, 0)
        return 0
    lax.fori_loop(0, NBLK, block, 0)

    pltpu.sync_copy(mxa.at[pl.ds(0, NPT)], mx_out.at[pl.ds(w * NPT, NPT)])
    pltpu.sync_copy(mna.at[pl.ds(0, NPT)], mn_out.at[pl.ds(w * NPT, NPT)])


BN = 2560


def _finalize_body(s_ref, s2_ref, mx_ref, mn_ref, cnt_ref, deg_ref,
                   hprev_ref, wp_ref, b_ref, out_ref):
    sm = s_ref[...]
    s2 = s2_ref[...]
    mx = mx_ref[...]
    mn = mn_ref[...]
    cnt = cnt_ref[:, 0:1]
    has = cnt > 0.0
    cnt_c = jnp.maximum(cnt, 1.0)
    mean = sm / cnt_c
    var = jnp.maximum(s2 / cnt_c - mean * mean, 0.0)
    std = jnp.sqrt(var + 1e-5)
    zero = jnp.zeros_like(mean)
    mean = jnp.where(has, mean, zero)
    std = jnp.where(has, std, jnp.full_like(std, float(np.sqrt(np.float32(1e-5)))))
    mx = jnp.where(has, mx, zero)
    mn = jnp.where(has, mn, zero)
    agg = jnp.concatenate([mean, mx, mn, std], axis=1)
    y = jnp.dot(agg, wp_ref[...], preferred_element_type=jnp.float32)
    deg = deg_ref[:, 0:1]
    ldeg = jnp.log(jnp.maximum(deg, 1.0) + 1.0)
    amp = ldeg * (1.0 / DELTA)
    att = DELTA / ldeg
    out = y[:, 0:128] + amp * y[:, 128:256] + att * y[:, 256:384]
    out_ref[...] = out + b_ref[...] + hprev_ref[...]


_finalize = pl.pallas_call(
    _finalize_body,
    grid=(NPAD // BN,),
    in_specs=[
        pl.BlockSpec((BN, 128), lambda i: (i, 0)),
        pl.BlockSpec((BN, 128), lambda i: (i, 0)),
        pl.BlockSpec((BN, 128), lambda i: (i, 0)),
        pl.BlockSpec((BN, 128), lambda i: (i, 0)),
        pl.BlockSpec((BN, 16), lambda i: (i, 0)),
        pl.BlockSpec((BN, 16), lambda i: (i, 0)),
        pl.BlockSpec((BN, 128), lambda i: (i, 0)),
        pl.BlockSpec((512, 384), lambda i: (0, 0)),
        pl.BlockSpec((1, 128), lambda i: (0, 0)),
    ],
    out_specs=pl.BlockSpec((BN, 128), lambda i: (i, 0)),
    out_shape=jax.ShapeDtypeStruct((NPAD, D), jnp.float32),
)


def kernel(x, edge_index, W0, b0, W1, b1, W2, b2):
    srcl, ldl, cnts, cnt_t, deg_t = _setup(edge_index[0], edge_index[1])
    cnt = cnt_t.reshape(-1)
    deg = deg_t.reshape(-1)
    cnt16 = jnp.broadcast_to(cnt[:, None], (NPAD, 16))
    deg16 = jnp.broadcast_to(deg[:, None], (NPAD, 16))
    zeros = jnp.zeros((NCORE, D), jnp.float32)
    h = jnp.concatenate([x, jnp.zeros((NPAD - N, D), x.dtype)], axis=0)
    for W, b in ((W0, b0), (W1, b1), (W2, b2)):
        sm, s2 = _scatter_sums(h, h * h, srcl, ldl, cnts, zeros)
        mx, mn = _scatter_minmax(h, srcl, ldl, cnts)
        wp = jnp.concatenate([W[0:512], W[512:1024], W[1024:1536]], axis=1)
        h = _finalize(sm, s2, mx, mn, cnt16, deg16, h, wp, b.reshape(1, 128))
    return h[:N]
